# baseline (speedup 1.0000x reference)
import functools

import jax
import jax.numpy as jnp
from jax import lax
from jax.experimental import pallas as pl
from jax.experimental.pallas import tpu as pltpu
from jax.experimental.pallas import tpu_sc as plsc

_N = 10
_L = 16
_ROWS, _COLS = 2048, 2048
_NC, _NS = 2, 16
_NW = _NC * _NS
_ROWS_W = _ROWS // _NW
_BR = 8
_NCHUNK = _ROWS_W // _BR


def _build_tables(xs_v, ys_v, sl_v, a_ref, b_ref):
    li = lax.iota(jnp.int32, _L)
    lo = jnp.clip(li - 1, 0, _N - 2)
    hi = lo + 1
    xs_lo = plsc.load_gather(xs_v, [lo])
    xs_hi = plsc.load_gather(xs_v, [hi])
    ys_lo = plsc.load_gather(ys_v, [lo])
    ys_hi = plsc.load_gather(ys_v, [hi])
    b = (ys_hi - ys_lo) / (xs_hi - xs_lo)
    a = ys_lo - xs_lo * b
    xs_vec, ys_vec, sl_vec = xs_v[...], ys_v[...], sl_v[...]

    def lane(v, k):
        return jnp.sum(jnp.where(li == k, v, 0.0))

    s0 = lane(sl_vec, 0)
    s1 = lane(sl_vec, 1)
    xs0 = lane(xs_vec, 0)
    ys0 = lane(ys_vec, 0)
    xs_last = lane(xs_vec, _N - 1)
    ys_last = lane(ys_vec, _N - 1)
    fz = jnp.zeros((_L,), jnp.float32)
    m_left = li == 0
    b = jnp.where(m_left, fz + s0, b)
    a = jnp.where(m_left, fz + (ys0 - xs0 * s0), a)
    m_right = li >= _N
    b = jnp.where(m_right, fz + s1, b)
    a = jnp.where(m_right, fz + (ys_last - xs_last * s1), a)
    a_ref[...] = a
    b_ref[...] = b
    inv_h = (fz + (_N - 1).__float__()) / (fz + (xs_last - xs0))
    c0 = 1.0 - xs0 * inv_h
    return inv_h, c0


def _sc_kernel(x_hbm, xs_hbm, ys_hbm, sl_hbm, out_hbm,
               xs_v, ys_v, sl_v, a_v, b_v,
               xb0, xb1, ob0, ob1, isem0, isem1, osem0, osem1):
    pltpu.sync_copy(xs_hbm, xs_v.at[pl.ds(0, _N)])
    pltpu.sync_copy(ys_hbm, ys_v.at[pl.ds(0, _N)])
    pltpu.sync_copy(sl_hbm, sl_v.at[pl.ds(0, 2)])
    inv_h, c0 = _build_tables(xs_v, ys_v, sl_v, a_v, b_v)
    top = jnp.zeros((_L,), jnp.float32) + _N.__float__()

    wid = lax.axis_index("s") * _NC + lax.axis_index("c")
    row0 = wid * _ROWS_W
    xb = (xb0, xb1)
    ob = (ob0, ob1)
    isem = (isem0, isem1)
    osem = (osem0, osem1)

    def compute(slot):
        xbuf, obuf = xb[slot], ob[slot]

        @plsc.parallel_loop(0, _BR * _COLS, step=_L, unroll=16)
        def _(j):
            r = j >> 11
            cc = j & (_COLS - 1)
            xv = xbuf[r, pl.ds(cc, _L)]
            t = xv * inv_h + c0
            t = jnp.minimum(jnp.maximum(t, 0.0), top)
            jj = t.astype(jnp.int32)
            av = plsc.load_gather(a_v, [jj])
            bv = plsc.load_gather(b_v, [jj])
            obuf[r, pl.ds(cc, _L)] = av + bv * xv

    def wait_in(slot):
        pltpu.make_async_copy(x_hbm.at[pl.ds(0, _BR), :], xb[slot],
                              isem[slot]).wait()

    def wait_out(slot):
        pltpu.make_async_copy(ob[slot], out_hbm.at[pl.ds(0, _BR), :],
                              osem[slot]).wait()

    pltpu.async_copy(x_hbm.at[pl.ds(row0, _BR), :], xb[0], isem[0])
    pltpu.async_copy(x_hbm.at[pl.ds(row0 + _BR, _BR), :], xb[1], isem[1])
    nc2 = _NCHUNK // 2

    def chunk_pair(c2, carry):
        for slot in (0, 1):
            c = 2 * c2 + slot
            wait_in(slot)

            @pl.when(c2 >= 1)
            def _():
                wait_out(slot)

            compute(slot)
            pltpu.async_copy(
                ob[slot], out_hbm.at[pl.ds(row0 + c * _BR, _BR), :],
                osem[slot])

            @pl.when(c2 < nc2 - 1)
            def _():
                pltpu.async_copy(
                    x_hbm.at[pl.ds(row0 + (c + 2) * _BR, _BR), :],
                    xb[slot], isem[slot])
        return carry

    lax.fori_loop(0, nc2, chunk_pair, 0)
    wait_out(0)
    wait_out(1)


@jax.jit
def _piecewise(x, xs, ys, slopes):
    mesh = plsc.VectorSubcoreMesh(core_axis_name="c", subcore_axis_name="s",
                                  num_cores=_NC)
    run = functools.partial(
        pl.kernel,
        mesh=mesh,
        compiler_params=pltpu.CompilerParams(needs_layout_passes=False),
        out_type=jax.ShapeDtypeStruct((_ROWS, _COLS), jnp.float32),
        scratch_types=[
            pltpu.VMEM((_L,), jnp.float32),
            pltpu.VMEM((_L,), jnp.float32),
            pltpu.VMEM((_L,), jnp.float32),
            pltpu.VMEM((_L,), jnp.float32),
            pltpu.VMEM((_L,), jnp.float32),
            pltpu.VMEM((_BR, _COLS), jnp.float32),
            pltpu.VMEM((_BR, _COLS), jnp.float32),
            pltpu.VMEM((_BR, _COLS), jnp.float32),
            pltpu.VMEM((_BR, _COLS), jnp.float32),
            pltpu.SemaphoreType.DMA,
            pltpu.SemaphoreType.DMA,
            pltpu.SemaphoreType.DMA,
            pltpu.SemaphoreType.DMA,
        ],
    )(_sc_kernel)
    return run(x, xs, ys, slopes)


def kernel(x, xs, ys, slopes):
    return _piecewise(x, xs, ys, slopes)


# speedup vs baseline: 1.2698x; 1.2698x over previous
import functools

import jax
import jax.numpy as jnp
from jax import lax
from jax.experimental import pallas as pl
from jax.experimental.pallas import tpu as pltpu
from jax.experimental.pallas import tpu_sc as plsc

_N = 10
_L = 16
_ROWS, _COLS = 2048, 2048
_NC, _NS = 2, 16
_NW = _NC * _NS
_ROWS_W = _ROWS // _NW
_BR = 8
_NCHUNK = _ROWS_W // _BR


def _build_tables(xs_v, ys_v, sl_v, a_ref, b_ref):
    li = lax.iota(jnp.int32, _L)
    lo = jnp.clip(li - 1, 0, _N - 2)
    hi = lo + 1
    xs_lo = plsc.load_gather(xs_v, [lo])
    xs_hi = plsc.load_gather(xs_v, [hi])
    ys_lo = plsc.load_gather(ys_v, [lo])
    ys_hi = plsc.load_gather(ys_v, [hi])
    b = (ys_hi - ys_lo) / (xs_hi - xs_lo)
    a = ys_lo - xs_lo * b
    xs_vec, ys_vec, sl_vec = xs_v[...], ys_v[...], sl_v[...]

    def lane(v, k):
        return jnp.sum(jnp.where(li == k, v, 0.0))

    s0 = lane(sl_vec, 0)
    s1 = lane(sl_vec, 1)
    xs0 = lane(xs_vec, 0)
    ys0 = lane(ys_vec, 0)
    xs_last = lane(xs_vec, _N - 1)
    ys_last = lane(ys_vec, _N - 1)
    fz = jnp.zeros((_L,), jnp.float32)
    m_left = li == 0
    b = jnp.where(m_left, fz + s0, b)
    a = jnp.where(m_left, fz + (ys0 - xs0 * s0), a)
    m_right = li >= _N
    b = jnp.where(m_right, fz + s1, b)
    a = jnp.where(m_right, fz + (ys_last - xs_last * s1), a)
    a_ref[...] = a
    b_ref[...] = b
    inv_h = (fz + (_N - 1).__float__()) / (fz + (xs_last - xs0))
    c0 = 1.0 - xs0 * inv_h
    return inv_h, c0


def _sc_kernel(x_hbm, xs_hbm, ys_hbm, sl_hbm, out_hbm,
               xs_v, ys_v, sl_v, a_v, b_v,
               xb0, xb1, ob0, ob1, isem0, isem1, osem0, osem1):
    pltpu.sync_copy(xs_hbm, xs_v.at[pl.ds(0, _N)])
    pltpu.sync_copy(ys_hbm, ys_v.at[pl.ds(0, _N)])
    pltpu.sync_copy(sl_hbm, sl_v.at[pl.ds(0, 2)])
    inv_h, c0 = _build_tables(xs_v, ys_v, sl_v, a_v, b_v)
    top = jnp.zeros((_L,), jnp.float32) + _N.__float__()

    wid = lax.axis_index("s") * _NC + lax.axis_index("c")
    row0 = wid * _ROWS_W
    xb = (xb0, xb1)
    ob = (ob0, ob1)
    isem = (isem0, isem1)
    osem = (osem0, osem1)

    def compute(slot):
        xbuf, obuf = xb[slot], ob[slot]

        @plsc.parallel_loop(0, _BR * _COLS, step=_L, unroll=8)
        def _(j):
            r = j >> 11
            cc = j & (_COLS - 1)
            xv = xbuf[r, pl.ds(cc, _L)]
            t = xv * inv_h + c0
            t = jnp.minimum(jnp.maximum(t, 0.0), top)
            jj = t.astype(jnp.int32)
            av = plsc.load_gather(a_v, [jj])
            bv = plsc.load_gather(b_v, [jj])
            obuf[r, pl.ds(cc, _L)] = av + bv * xv

    def wait_in(slot):
        pltpu.make_async_copy(x_hbm.at[pl.ds(0, _BR), :], xb[slot],
                              isem[slot]).wait()

    def wait_out(slot):
        pltpu.make_async_copy(ob[slot], out_hbm.at[pl.ds(0, _BR), :],
                              osem[slot]).wait()

    pltpu.async_copy(x_hbm.at[pl.ds(row0, _BR), :], xb[0], isem[0])
    pltpu.async_copy(x_hbm.at[pl.ds(row0 + _BR, _BR), :], xb[1], isem[1])
    nc2 = _NCHUNK // 2

    def chunk_pair(c2, carry):
        for slot in (0, 1):
            c = 2 * c2 + slot
            wait_in(slot)

            @pl.when(c2 >= 1)
            def _():
                wait_out(slot)

            compute(slot)
            pltpu.async_copy(
                ob[slot], out_hbm.at[pl.ds(row0 + c * _BR, _BR), :],
                osem[slot])

            @pl.when(c2 < nc2 - 1)
            def _():
                pltpu.async_copy(
                    x_hbm.at[pl.ds(row0 + (c + 2) * _BR, _BR), :],
                    xb[slot], isem[slot])
        return carry

    lax.fori_loop(0, nc2, chunk_pair, 0)
    wait_out(0)
    wait_out(1)


@jax.jit
def _piecewise(x, xs, ys, slopes):
    mesh = plsc.VectorSubcoreMesh(core_axis_name="c", subcore_axis_name="s",
                                  num_cores=_NC)
    run = functools.partial(
        pl.kernel,
        mesh=mesh,
        compiler_params=pltpu.CompilerParams(needs_layout_passes=False),
        out_type=jax.ShapeDtypeStruct((_ROWS, _COLS), jnp.float32),
        scratch_types=[
            pltpu.VMEM((_L,), jnp.float32),
            pltpu.VMEM((_L,), jnp.float32),
            pltpu.VMEM((_L,), jnp.float32),
            pltpu.VMEM((_L,), jnp.float32),
            pltpu.VMEM((_L,), jnp.float32),
            pltpu.VMEM((_BR, _COLS), jnp.float32),
            pltpu.VMEM((_BR, _COLS), jnp.float32),
            pltpu.VMEM((_BR, _COLS), jnp.float32),
            pltpu.VMEM((_BR, _COLS), jnp.float32),
            pltpu.SemaphoreType.DMA,
            pltpu.SemaphoreType.DMA,
            pltpu.SemaphoreType.DMA,
            pltpu.SemaphoreType.DMA,
        ],
    )(_sc_kernel)
    return run(x, xs, ys, slopes)


def kernel(x, xs, ys, slopes):
    return _piecewise(x, xs, ys, slopes)


# speedup vs baseline: 1.3067x; 1.0291x over previous
import functools

import jax
import jax.numpy as jnp
from jax import lax
from jax.experimental import pallas as pl
from jax.experimental.pallas import tpu as pltpu
from jax.experimental.pallas import tpu_sc as plsc

_N = 10
_L = 16
_ROWS, _COLS = 2048, 2048
_NC, _NS = 2, 16
_NW = _NC * _NS
_ROWS_W = _ROWS // _NW
_BR = 8
_NCHUNK = _ROWS_W // _BR


def _build_tables(xs_v, ys_v, sl_v, a_ref, b_ref):
    li = lax.iota(jnp.int32, _L)
    lo = jnp.clip(li - 1, 0, _N - 2)
    hi = lo + 1
    xs_lo = plsc.load_gather(xs_v, [lo])
    xs_hi = plsc.load_gather(xs_v, [hi])
    ys_lo = plsc.load_gather(ys_v, [lo])
    ys_hi = plsc.load_gather(ys_v, [hi])
    b = (ys_hi - ys_lo) / (xs_hi - xs_lo)
    a = ys_lo - xs_lo * b
    xs_vec, ys_vec, sl_vec = xs_v[...], ys_v[...], sl_v[...]

    def lane(v, k):
        return jnp.sum(jnp.where(li == k, v, 0.0))

    s0 = lane(sl_vec, 0)
    s1 = lane(sl_vec, 1)
    xs0 = lane(xs_vec, 0)
    ys0 = lane(ys_vec, 0)
    xs_last = lane(xs_vec, _N - 1)
    ys_last = lane(ys_vec, _N - 1)
    fz = jnp.zeros((_L,), jnp.float32)
    m_left = li == 0
    b = jnp.where(m_left, fz + s0, b)
    a = jnp.where(m_left, fz + (ys0 - xs0 * s0), a)
    m_right = li >= _N
    b = jnp.where(m_right, fz + s1, b)
    a = jnp.where(m_right, fz + (ys_last - xs_last * s1), a)
    a_ref[...] = a
    b_ref[...] = b
    inv_h = (fz + (_N - 1).__float__()) / (fz + (xs_last - xs0))
    c0 = 1.0 - xs0 * inv_h
    return inv_h, c0


def _sc_kernel(x_hbm, xs_hbm, ys_hbm, sl_hbm, out_hbm,
               xs_v, ys_v, sl_v, a_v, b_v,
               xb0, xb1, ob0, ob1, isem0, isem1, osem0, osem1):
    wid = lax.axis_index("s") * _NC + lax.axis_index("c")
    row0 = wid * _ROWS_W
    xb = (xb0, xb1)
    ob = (ob0, ob1)
    isem = (isem0, isem1)
    osem = (osem0, osem1)

    pltpu.async_copy(x_hbm.at[pl.ds(row0, _BR), :], xb[0], isem[0])
    pltpu.async_copy(x_hbm.at[pl.ds(row0 + _BR, _BR), :], xb[1], isem[1])

    pltpu.sync_copy(xs_hbm, xs_v.at[pl.ds(0, _N)])
    pltpu.sync_copy(ys_hbm, ys_v.at[pl.ds(0, _N)])
    pltpu.sync_copy(sl_hbm, sl_v.at[pl.ds(0, 2)])
    inv_h, c0 = _build_tables(xs_v, ys_v, sl_v, a_v, b_v)
    top = jnp.zeros((_L,), jnp.float32) + _N.__float__()

    def compute(slot):
        xbuf, obuf = xb[slot], ob[slot]

        @plsc.parallel_loop(0, _BR * _COLS, step=_L, unroll=8)
        def _(j):
            r = j >> 11
            cc = j & (_COLS - 1)
            xv = xbuf[r, pl.ds(cc, _L)]
            t = xv * inv_h + c0
            t = jnp.minimum(jnp.maximum(t, 0.0), top)
            jj = t.astype(jnp.int32)
            av = plsc.load_gather(a_v, [jj])
            bv = plsc.load_gather(b_v, [jj])
            obuf[r, pl.ds(cc, _L)] = av + bv * xv

    def wait_in(slot):
        pltpu.make_async_copy(x_hbm.at[pl.ds(0, _BR), :], xb[slot],
                              isem[slot]).wait()

    def wait_out(slot):
        pltpu.make_async_copy(ob[slot], out_hbm.at[pl.ds(0, _BR), :],
                              osem[slot]).wait()

    nc2 = _NCHUNK // 2

    def chunk_pair(c2, carry):
        for slot in (0, 1):
            c = 2 * c2 + slot
            wait_in(slot)

            @pl.when(c2 >= 1)
            def _():
                wait_out(slot)

            compute(slot)
            pltpu.async_copy(
                ob[slot], out_hbm.at[pl.ds(row0 + c * _BR, _BR), :],
                osem[slot])

            @pl.when(c2 < nc2 - 1)
            def _():
                pltpu.async_copy(
                    x_hbm.at[pl.ds(row0 + (c + 2) * _BR, _BR), :],
                    xb[slot], isem[slot])
        return carry

    lax.fori_loop(0, nc2, chunk_pair, 0)
    wait_out(0)
    wait_out(1)


@jax.jit
def _piecewise(x, xs, ys, slopes):
    mesh = plsc.VectorSubcoreMesh(core_axis_name="c", subcore_axis_name="s",
                                  num_cores=_NC)
    run = functools.partial(
        pl.kernel,
        mesh=mesh,
        compiler_params=pltpu.CompilerParams(needs_layout_passes=False),
        out_type=jax.ShapeDtypeStruct((_ROWS, _COLS), jnp.float32),
        scratch_types=[
            pltpu.VMEM((_L,), jnp.float32),
            pltpu.VMEM((_L,), jnp.float32),
            pltpu.VMEM((_L,), jnp.float32),
            pltpu.VMEM((_L,), jnp.float32),
            pltpu.VMEM((_L,), jnp.float32),
            pltpu.VMEM((_BR, _COLS), jnp.float32),
            pltpu.VMEM((_BR, _COLS), jnp.float32),
            pltpu.VMEM((_BR, _COLS), jnp.float32),
            pltpu.VMEM((_BR, _COLS), jnp.float32),
            pltpu.SemaphoreType.DMA,
            pltpu.SemaphoreType.DMA,
            pltpu.SemaphoreType.DMA,
            pltpu.SemaphoreType.DMA,
        ],
    )(_sc_kernel)
    return run(x, xs, ys, slopes)


def kernel(x, xs, ys, slopes):
    return _piecewise(x, xs, ys, slopes)
